# baseline (speedup 1.0000x reference)
import functools

import jax
import jax.numpy as jnp
from jax import lax
from jax.experimental import pallas as pl
from jax.experimental.pallas import tpu as pltpu
from jax.experimental.pallas import tpu_sc as plsc

K = 256
B = 128
N = 32768
L = 16
NB = 256
NVEC = N // L
NWORKERS = 32
ROWS_PER_W = B // NWORKERS
U = 8

PASSES = ((None, 23, 8), (23, 15, 8), (15, 7, 8), (7, 0, 7))


def _topk_body(x_hbm, out_hbm, row_v, hist_v):
    cid = lax.axis_index("c")
    sid = lax.axis_index("s")
    wid = sid * 2 + cid

    lane = lax.broadcasted_iota(jnp.int32, (L,), 0)
    ones = jnp.ones((L,), jnp.int32)
    zeros = jnp.zeros((L,), jnp.int32)
    mask31 = jnp.int32(0x7FFFFFFF)

    def zero_hist():
        def zbody(i, _):
            for u in range(U):
                hist_v[pl.ds((i * U + u) * L, L)] = zeros
            return 0
        lax.fori_loop(0, NB * L // L // U, zbody, 0)

    def hist_pass(pshift, dshift, dbits, prefix):
        dmask = jnp.int32((1 << dbits) - 1)

        def body(i, _):
            for u in range(U):
                v = row_v[pl.ds((i * U + u) * L, L)]
                a = lax.bitwise_and(v, mask31)
                d = lax.bitwise_and(lax.shift_right_logical(a, dshift), dmask)
                idx = d * L + lane
                if pshift is None:
                    plsc.addupdate_scatter(hist_v, [idx], ones)
                else:
                    m = lax.shift_right_logical(a, pshift) == prefix
                    plsc.addupdate_scatter(hist_v, [idx], ones, mask=m)
            return 0
        lax.fori_loop(0, NVEC // U, body, 0)

    def scan_hist(dbits, kin):
        nch = (1 << dbits) // L
        cum = jnp.int32(0)
        found = jnp.int32(0)
        need_star = jnp.int32(0)
        t_star = jnp.int32(0)
        cs_star = zeros
        rc_star = zeros
        for t in range(nch - 1, -1, -1):
            acc = zeros
            base = lane * L + t * L * L
            for w in range(L):
                acc = acc + plsc.load_gather(hist_v, [base + w])
            rc = lax.rev(acc, (0,))
            cs = plsc.cumsum(rc)
            tot = jnp.max(cs)
            hit = jnp.logical_and(found == 0, cum + tot >= kin)
            hb = (zeros + hit.astype(jnp.int32)) == 1
            need_star = jnp.where(hit, kin - cum, need_star)
            t_star = jnp.where(hit, t, t_star)
            cs_star = jnp.where(hb, cs, cs_star)
            rc_star = jnp.where(hb, rc, rc_star)
            found = jnp.where(hit, 1, found)
            cum = cum + tot
        jj = jnp.max(plsc.all_reduce_ffs(cs_star >= need_star))
        cs_at = jnp.sum(jnp.where(lane == jj, cs_star, 0))
        cnt_b = jnp.sum(jnp.where(lane == jj, rc_star, 0))
        digit = t_star * L + (L - 1) - jj
        kin_next = need_star - (cs_at - cnt_b)
        return digit, kin_next

    def process_row(j, _carry):
        r = wid * ROWS_PER_W + j
        pltpu.sync_copy(x_hbm.at[r], row_v)

        prefix = jnp.int32(0)
        kin = jnp.int32(K)
        for (pshift, dshift, dbits) in PASSES:
            zero_hist()
            hist_pass(pshift, dshift, dbits, prefix)
            digit, kin = scan_hist(dbits, kin)
            prefix = (prefix << dbits) | digit
        T = prefix

        def obody(i, _):
            for u in range(U):
                v = row_v[pl.ds((i * U + u) * L, L)]
                a = lax.bitwise_and(v, mask31)
                row_v[pl.ds((i * U + u) * L, L)] = jnp.where(a >= T, a, 0)
            return 0
        lax.fori_loop(0, NVEC // U, obody, 0)
        pltpu.sync_copy(row_v, out_hbm.at[r])
        return 0

    lax.fori_loop(0, ROWS_PER_W, process_row, 0)


@jax.jit
def _topk_mask(bits):
    mesh = plsc.VectorSubcoreMesh(core_axis_name="c", subcore_axis_name="s")
    f = functools.partial(
        pl.kernel,
        out_type=jax.ShapeDtypeStruct((B, N), jnp.int32),
        mesh=mesh,
        scratch_types=[
            pltpu.VMEM((N,), jnp.int32),
            pltpu.VMEM((NB * L,), jnp.int32),
        ],
        compiler_params=pltpu.CompilerParams(needs_layout_passes=False),
    )(_topk_body)
    return f(bits)


def kernel(inputs):
    bits = lax.bitcast_convert_type(inputs, jnp.int32)
    out_bits = _topk_mask(bits)
    return lax.bitcast_convert_type(out_bits, jnp.float32)


# speedup vs baseline: 2.4948x; 2.4948x over previous
import functools

import jax
import jax.numpy as jnp
from jax import lax
from jax.experimental import pallas as pl
from jax.experimental.pallas import tpu as pltpu
from jax.experimental.pallas import tpu_sc as plsc

K = 256
B = 128
N = 32768
L = 16
NB = 256
NCH = NB // L
NVEC = N // L
NWORKERS = 32
ROWS_PER_W = B // NWORKERS
U = 8

PASSES = ((None, 23, 8), (23, 15, 8), (15, 7, 8), (7, 0, 7))


def _tree_sum(vs):
    while len(vs) > 1:
        vs = [vs[k] + vs[k + 1] for k in range(0, len(vs), 2)] + (
            [vs[-1]] if len(vs) % 2 else [])
    return vs[0]


def _topk_body(x_hbm, out_hbm, row_v, hist_v, scr_v):
    cid = lax.axis_index("c")
    sid = lax.axis_index("s")
    wid = sid * 2 + cid

    lane = lax.broadcasted_iota(jnp.int32, (L,), 0)
    lane16 = lane * L
    ones = jnp.ones((L,), jnp.int32)
    zeros = jnp.zeros((L,), jnp.int32)
    mask31 = jnp.int32(0x7FFFFFFF)

    def zero_hist():
        def zbody(i, _):
            for u in range(U):
                hist_v[pl.ds((i * U + u) * L, L)] = zeros
            return 0
        lax.fori_loop(0, NB * L // L // U, zbody, 0)

    def hist_pass(pshift, dshift, dbits, prefix):
        dmask = jnp.int32((1 << dbits) - 1)

        def body(i, _):
            vs = [row_v[pl.ds((i * U + u) * L, L)] for u in range(U)]
            aa = [lax.bitwise_and(v, mask31) for v in vs]
            dd = [lax.bitwise_and(lax.shift_right_logical(a, dshift), dmask)
                  for a in aa]
            ii = [d * L + lane for d in dd]
            if pshift is None:
                for u in range(U):
                    plsc.addupdate_scatter(hist_v, [ii[u]], ones)
            else:
                mm = [lax.shift_right_logical(a, pshift) == prefix for a in aa]
                for u in range(U):
                    plsc.addupdate_scatter(hist_v, [ii[u]], ones, mask=mm[u])
            return 0
        lax.fori_loop(0, NVEC // U, body, 0)

    def scan_hist(kin):
        for t in range(NCH):
            g = [plsc.load_gather(hist_v, [lane16 + (t * NB + w)])
                 for w in range(L)]
            acc = _tree_sum(g)
            rc = lax.rev(acc, (0,))
            scr_v[pl.ds(t * L, L)] = plsc.cumsum(rc)
        totals = plsc.load_gather(scr_v, [lane16 + (L - 1)])
        ct = plsc.cumsum(lax.rev(totals, (0,)))
        q = jnp.max(plsc.all_reduce_ffs(ct >= kin))
        cumab = jnp.sum(jnp.where(lane == q - 1, ct, 0))
        tstar = L - 1 - q
        need = kin - cumab
        cs_star = scr_v[pl.ds(tstar * L, L)]
        jj = jnp.max(plsc.all_reduce_ffs(cs_star >= need))
        above = jnp.sum(jnp.where(lane == jj - 1, cs_star, 0))
        digit = tstar * L + (L - 1) - jj
        kin_next = need - above
        return digit, kin_next

    def process_row(j, _carry):
        r = wid * ROWS_PER_W + j
        pltpu.sync_copy(x_hbm.at[r], row_v)

        prefix = jnp.int32(0)
        kin = jnp.int32(K)
        for (pshift, dshift, dbits) in PASSES:
            zero_hist()
            hist_pass(pshift, dshift, dbits, prefix)
            digit, kin = scan_hist(kin)
            prefix = (prefix << dbits) | digit
        T = prefix

        def obody(i, _):
            vs = [row_v[pl.ds((i * U + u) * L, L)] for u in range(U)]
            aa = [lax.bitwise_and(v, mask31) for v in vs]
            oo = [jnp.where(a >= T, a, 0) for a in aa]
            for u in range(U):
                row_v[pl.ds((i * U + u) * L, L)] = oo[u]
            return 0
        lax.fori_loop(0, NVEC // U, obody, 0)
        pltpu.sync_copy(row_v, out_hbm.at[r])
        return 0

    lax.fori_loop(0, ROWS_PER_W, process_row, 0)


@jax.jit
def _topk_mask(bits):
    mesh = plsc.VectorSubcoreMesh(core_axis_name="c", subcore_axis_name="s")
    f = functools.partial(
        pl.kernel,
        out_type=jax.ShapeDtypeStruct((B, N), jnp.int32),
        mesh=mesh,
        scratch_types=[
            pltpu.VMEM((N,), jnp.int32),
            pltpu.VMEM((NB * L,), jnp.int32),
            pltpu.VMEM((NB,), jnp.int32),
        ],
        compiler_params=pltpu.CompilerParams(needs_layout_passes=False),
    )(_topk_body)
    return f(bits)


def kernel(inputs):
    bits = lax.bitcast_convert_type(inputs, jnp.int32)
    out_bits = _topk_mask(bits)
    return lax.bitcast_convert_type(out_bits, jnp.float32)


# speedup vs baseline: 2.5422x; 1.0190x over previous
import functools

import jax
import jax.numpy as jnp
from jax import lax
from jax.experimental import pallas as pl
from jax.experimental.pallas import tpu as pltpu
from jax.experimental.pallas import tpu_sc as plsc

K = 256
B = 128
N = 32768
L = 16
NB = 256
NCH = NB // L
NVEC = N // L
NWORKERS = 32
ROWS_PER_W = B // NWORKERS
U = 16

PASSES = ((None, 23, 8), (23, 15, 8), (15, 7, 8), (7, 0, 7))


def _tree_sum(vs):
    while len(vs) > 1:
        vs = [vs[k] + vs[k + 1] for k in range(0, len(vs), 2)] + (
            [vs[-1]] if len(vs) % 2 else [])
    return vs[0]


def _topk_body(x_hbm, out_hbm, row_v, hist_v, scr_v):
    cid = lax.axis_index("c")
    sid = lax.axis_index("s")
    wid = sid * 2 + cid

    lane = lax.broadcasted_iota(jnp.int32, (L,), 0)
    lane16 = lane * L
    ones = jnp.ones((L,), jnp.int32)
    zeros = jnp.zeros((L,), jnp.int32)
    mask31 = jnp.int32(0x7FFFFFFF)

    def zero_hist():
        def zbody(i, _):
            for u in range(U):
                hist_v[pl.ds((i * U + u) * L, L)] = zeros
            return 0
        lax.fori_loop(0, NB * L // L // U, zbody, 0)

    def hist_pass(pshift, dshift, dbits, prefix):
        dmask = jnp.int32((1 << dbits) - 1)

        def body(i, _):
            vs = [row_v[pl.ds((i * U + u) * L, L)] for u in range(U)]
            aa = [lax.bitwise_and(v, mask31) for v in vs]
            dd = [lax.bitwise_and(lax.shift_right_logical(a, dshift), dmask)
                  for a in aa]
            ii = [d * L + lane for d in dd]
            if pshift is None:
                for u in range(U):
                    plsc.addupdate_scatter(hist_v, [ii[u]], ones)
            else:
                mm = [lax.shift_right_logical(a, pshift) == prefix for a in aa]
                for u in range(U):
                    plsc.addupdate_scatter(hist_v, [ii[u]], ones, mask=mm[u])
            return 0
        lax.fori_loop(0, NVEC // U, body, 0)

    def scan_hist(kin):
        for t in range(NCH):
            g = [plsc.load_gather(hist_v, [lane16 + (t * NB + w)])
                 for w in range(L)]
            acc = _tree_sum(g)
            rc = lax.rev(acc, (0,))
            scr_v[pl.ds(t * L, L)] = plsc.cumsum(rc)
        totals = plsc.load_gather(scr_v, [lane16 + (L - 1)])
        ct = plsc.cumsum(lax.rev(totals, (0,)))
        q = jnp.max(plsc.all_reduce_ffs(ct >= kin))
        cumab = jnp.sum(jnp.where(lane == q - 1, ct, 0))
        tstar = L - 1 - q
        need = kin - cumab
        cs_star = scr_v[pl.ds(tstar * L, L)]
        jj = jnp.max(plsc.all_reduce_ffs(cs_star >= need))
        above = jnp.sum(jnp.where(lane == jj - 1, cs_star, 0))
        digit = tstar * L + (L - 1) - jj
        kin_next = need - above
        return digit, kin_next

    def process_row(j, _carry):
        r = wid * ROWS_PER_W + j
        pltpu.sync_copy(x_hbm.at[r], row_v)

        prefix = jnp.int32(0)
        kin = jnp.int32(K)
        for (pshift, dshift, dbits) in PASSES:
            zero_hist()
            hist_pass(pshift, dshift, dbits, prefix)
            digit, kin = scan_hist(kin)
            prefix = (prefix << dbits) | digit
        T = prefix

        def obody(i, _):
            vs = [row_v[pl.ds((i * U + u) * L, L)] for u in range(U)]
            aa = [lax.bitwise_and(v, mask31) for v in vs]
            oo = [jnp.where(a >= T, a, 0) for a in aa]
            for u in range(U):
                row_v[pl.ds((i * U + u) * L, L)] = oo[u]
            return 0
        lax.fori_loop(0, NVEC // U, obody, 0)
        pltpu.sync_copy(row_v, out_hbm.at[r])
        return 0

    lax.fori_loop(0, ROWS_PER_W, process_row, 0)


@jax.jit
def _topk_mask(bits):
    mesh = plsc.VectorSubcoreMesh(core_axis_name="c", subcore_axis_name="s")
    f = functools.partial(
        pl.kernel,
        out_type=jax.ShapeDtypeStruct((B, N), jnp.int32),
        mesh=mesh,
        scratch_types=[
            pltpu.VMEM((N,), jnp.int32),
            pltpu.VMEM((NB * L,), jnp.int32),
            pltpu.VMEM((NB,), jnp.int32),
        ],
        compiler_params=pltpu.CompilerParams(needs_layout_passes=False),
    )(_topk_body)
    return f(bits)


def kernel(inputs):
    bits = lax.bitcast_convert_type(inputs, jnp.int32)
    out_bits = _topk_mask(bits)
    return lax.bitcast_convert_type(out_bits, jnp.float32)


# speedup vs baseline: 2.5721x; 1.0118x over previous
import functools

import jax
import jax.numpy as jnp
from jax import lax
from jax.experimental import pallas as pl
from jax.experimental.pallas import tpu as pltpu
from jax.experimental.pallas import tpu_sc as plsc

K = 256
B = 128
N = 32768
L = 16
NB = 256
NCH = NB // L
NVEC = N // L
NWORKERS = 32
ROWS_PER_W = B // NWORKERS
U = 16

SH_P = (31, 23, 15, 7)
SH_D = (23, 15, 7, 0)
DM = (255, 255, 255, 127)
DBITS = (8, 8, 8, 7)


def _tree_sum(vs):
    while len(vs) > 1:
        vs = [vs[k] + vs[k + 1] for k in range(0, len(vs), 2)] + (
            [vs[-1]] if len(vs) % 2 else [])
    return vs[0]


def _sel4(p, consts):
    v = jnp.int32(consts[3])
    for q in (2, 1, 0):
        v = jnp.where(p == q, jnp.int32(consts[q]), v)
    return v


def _topk_body(x_hbm, out_hbm, b0, b1, b2, hist_v, scr_v,
               si0, si1, si2, so0, so1, so2):
    bufs = (b0, b1, b2)
    isems = (si0, si1, si2)
    osems = (so0, so1, so2)

    cid = lax.axis_index("c")
    sid = lax.axis_index("s")
    wid = sid * 2 + cid
    base = wid * ROWS_PER_W

    lane = lax.broadcasted_iota(jnp.int32, (L,), 0)
    lane16 = lane * L
    ones = jnp.ones((L,), jnp.int32)
    zeros = jnp.zeros((L,), jnp.int32)
    mask31 = jnp.int32(0x7FFFFFFF)

    def zero_hist():
        def zbody(i, _):
            for u in range(U):
                hist_v[pl.ds((i * U + u) * L, L)] = zeros
            return 0
        lax.fori_loop(0, NB * L // L // U, zbody, 0)

    def hist_pass(row_v, pshift, dshift, dmask, prefix):
        def body(i, _):
            vs = [row_v[pl.ds((i * U + u) * L, L)] for u in range(U)]
            aa = [lax.bitwise_and(v, mask31) for v in vs]
            dd = [lax.bitwise_and(lax.shift_right_logical(a, dshift), dmask)
                  for a in aa]
            ii = [lax.bitwise_or(lax.shift_left(d, 4), lane) for d in dd]
            mm = [lax.shift_right_logical(a, pshift) == prefix for a in aa]
            for u in range(U):
                plsc.addupdate_scatter(hist_v, [ii[u]], ones, mask=mm[u])
            return 0
        lax.fori_loop(0, NVEC // U, body, 0)

    def scan_hist(kin):
        for t in range(NCH):
            g = [plsc.load_gather(hist_v, [lane16 + (t * NB + w)])
                 for w in range(L)]
            acc = _tree_sum(g)
            rc = lax.rev(acc, (0,))
            scr_v[pl.ds(t * L, L)] = plsc.cumsum(rc)
        totals = plsc.load_gather(scr_v, [lane16 + (L - 1)])
        ct = plsc.cumsum(lax.rev(totals, (0,)))
        q = jnp.max(plsc.all_reduce_ffs(ct >= kin))
        cumab = jnp.sum(jnp.where(lane == q - 1, ct, 0))
        tstar = L - 1 - q
        need = kin - cumab
        cs_star = scr_v[pl.ds(tstar * L, L)]
        jj = jnp.max(plsc.all_reduce_ffs(cs_star >= need))
        above = jnp.sum(jnp.where(lane == jj - 1, cs_star, 0))
        digit = tstar * L + (L - 1) - jj
        kin_next = need - above
        return digit, kin_next

    def compute_threshold(row_v):
        def pass_body(p, carry):
            prefix, kin = carry
            pshift = _sel4(p, SH_P)
            dshift = _sel4(p, SH_D)
            dmask = _sel4(p, DM)
            dbits = _sel4(p, DBITS)
            zero_hist()
            hist_pass(row_v, pshift, dshift, dmask, prefix)
            digit, kin = scan_hist(kin)
            prefix = lax.bitwise_or(lax.shift_left(prefix, dbits), digit)
            return (prefix, kin)
        T, _ = lax.fori_loop(0, 4, pass_body, (jnp.int32(0), jnp.int32(K)))
        return T

    def output_pass(row_v, T):
        def obody(i, _):
            vs = [row_v[pl.ds((i * U + u) * L, L)] for u in range(U)]
            aa = [lax.bitwise_and(v, mask31) for v in vs]
            oo = [jnp.where(a >= T, a, 0) for a in aa]
            for u in range(U):
                row_v[pl.ds((i * U + u) * L, L)] = oo[u]
            return 0
        lax.fori_loop(0, NVEC // U, obody, 0)

    def in_copy(j, buf):
        return pltpu.make_async_copy(x_hbm.at[base + j], buf, isems[j % 3])

    def out_copy(j, buf):
        return pltpu.make_async_copy(buf, out_hbm.at[base + j], osems[j % 3])

    for m in range(3):
        in_copy(m, bufs[m]).start()

    for j in range(ROWS_PER_W):
        bj = bufs[j % 3]
        in_copy(j, bj).wait()
        T = compute_threshold(bj)
        if j == 1:
            out_copy(0, bufs[0]).wait()
            in_copy(3, bufs[0]).start()
        output_pass(bj, T)
        out_copy(j, bj).start()

    for j in (1, 2, 3):
        out_copy(j, bufs[j % 3]).wait()


@jax.jit
def _topk_mask(bits):
    mesh = plsc.VectorSubcoreMesh(core_axis_name="c", subcore_axis_name="s")
    f = functools.partial(
        pl.kernel,
        out_type=jax.ShapeDtypeStruct((B, N), jnp.int32),
        mesh=mesh,
        scratch_types=[
            pltpu.VMEM((N,), jnp.int32),
            pltpu.VMEM((N,), jnp.int32),
            pltpu.VMEM((N,), jnp.int32),
            pltpu.VMEM((NB * L,), jnp.int32),
            pltpu.VMEM((NB,), jnp.int32),
            pltpu.SemaphoreType.DMA,
            pltpu.SemaphoreType.DMA,
            pltpu.SemaphoreType.DMA,
            pltpu.SemaphoreType.DMA,
            pltpu.SemaphoreType.DMA,
            pltpu.SemaphoreType.DMA,
        ],
        compiler_params=pltpu.CompilerParams(needs_layout_passes=False),
    )(_topk_body)
    return f(bits)


def kernel(inputs):
    bits = lax.bitcast_convert_type(inputs, jnp.int32)
    out_bits = _topk_mask(bits)
    return lax.bitcast_convert_type(out_bits, jnp.float32)


# speedup vs baseline: 3.7278x; 1.4493x over previous
import functools

import jax
import jax.numpy as jnp
from jax import lax
from jax.experimental import pallas as pl
from jax.experimental.pallas import tpu as pltpu
from jax.experimental.pallas import tpu_sc as plsc

K = 256
B = 128
N = 32768
L = 16
NB = 2048
NCH = NB // L
NGRP = NCH // L
NVEC = N // L
NWORKERS = 32
ROWS_PER_W = B // NWORKERS
U = 16

NPASS = 3
SH_P = (31, 20, 9)
SH_D = (20, 9, 0)
DM = (2047, 2047, 511)
DBITS = (11, 11, 9)


def _sel(p, consts):
    v = jnp.int32(consts[-1])
    for q in range(len(consts) - 2, -1, -1):
        v = jnp.where(p == q, jnp.int32(consts[q]), v)
    return v


def _topk_body(x_hbm, out_hbm, b0, b1, b2, hist_v, scr_v,
               si0, si1, si2, so0, so1, so2):
    bufs = (b0, b1, b2)
    isems = (si0, si1, si2)
    osems = (so0, so1, so2)

    cid = lax.axis_index("c")
    sid = lax.axis_index("s")
    wid = sid * 2 + cid
    base = wid * ROWS_PER_W

    lane = lax.broadcasted_iota(jnp.int32, (L,), 0)
    lane16 = lane * L
    ones = jnp.ones((L,), jnp.int32)
    zeros = jnp.zeros((L,), jnp.int32)
    mask31 = jnp.int32(0x7FFFFFFF)

    def extract(vec, i):
        return jnp.sum(jnp.where(lane == i, vec, 0))

    def zero_hist():
        def zbody(i, _):
            for u in range(U):
                hist_v[pl.ds((i * U + u) * L, L)] = zeros
            return 0
        lax.fori_loop(0, NB // L // U, zbody, 0)

    def hist_pass(row_v, pshift, dshift, dmask, prefix):
        def body(i, _):
            vs = [row_v[pl.ds((i * U + u) * L, L)] for u in range(U)]
            aa = [lax.bitwise_and(v, mask31) for v in vs]
            dd = [lax.bitwise_and(lax.shift_right_logical(a, dshift), dmask)
                  for a in aa]
            mm = [lax.shift_right_logical(a, pshift) == prefix for a in aa]
            for u in range(U):
                plsc.addupdate_scatter(hist_v, [dd[u]], ones, mask=mm[u])
            return 0
        lax.fori_loop(0, NVEC // U, body, 0)

    def scan_hist(kin):
        def sbody(i, _):
            cs = [plsc.cumsum(lax.rev(hist_v[pl.ds((i * 8 + u) * L, L)], (0,)))
                  for u in range(8)]
            for u in range(8):
                scr_v[pl.ds((i * 8 + u) * L, L)] = cs[u]
            return 0
        lax.fori_loop(0, NCH // 8, sbody, 0)
        tots = [plsc.load_gather(scr_v, [lane16 + (g * 256 + (L - 1))])
                for g in range(NGRP)]
        csg = [plsc.cumsum(lax.rev(t, (0,))) for t in tots]
        gts = [jnp.max(c) for c in csg]
        cum = jnp.int32(0)
        found = jnp.int32(0)
        gstar = jnp.int32(0)
        need_g = jnp.int32(0)
        cs_g = zeros
        for g in range(NGRP - 1, -1, -1):
            hit = jnp.logical_and(found == 0, cum + gts[g] >= kin)
            hb = (zeros + hit.astype(jnp.int32)) == 1
            gstar = jnp.where(hit, g, gstar)
            need_g = jnp.where(hit, kin - cum, need_g)
            cs_g = jnp.where(hb, csg[g], cs_g)
            found = jnp.where(hit, 1, found)
            cum = cum + gts[g]
        q1 = jnp.max(plsc.all_reduce_ffs(cs_g >= need_g))
        tstar = gstar * L + (L - 1) - q1
        need_c = need_g - extract(cs_g, q1 - 1)
        cs_star = scr_v[pl.ds(tstar * L, L)]
        q2 = jnp.max(plsc.all_reduce_ffs(cs_star >= need_c))
        digit = tstar * L + (L - 1) - q2
        kin_next = need_c - extract(cs_star, q2 - 1)
        return digit, kin_next

    def compute_threshold(row_v):
        def pass_body(p, carry):
            prefix, kin = carry
            pshift = _sel(p, SH_P)
            dshift = _sel(p, SH_D)
            dmask = _sel(p, DM)
            dbits = _sel(p, DBITS)
            zero_hist()
            hist_pass(row_v, pshift, dshift, dmask, prefix)
            digit, kin = scan_hist(kin)
            prefix = lax.bitwise_or(lax.shift_left(prefix, dbits), digit)
            return (prefix, kin)
        T, _ = lax.fori_loop(0, NPASS, pass_body,
                             (jnp.int32(0), jnp.int32(K)))
        return T

    def output_pass(row_v, T):
        def obody(i, _):
            vs = [row_v[pl.ds((i * U + u) * L, L)] for u in range(U)]
            aa = [lax.bitwise_and(v, mask31) for v in vs]
            oo = [jnp.where(a >= T, a, 0) for a in aa]
            for u in range(U):
                row_v[pl.ds((i * U + u) * L, L)] = oo[u]
            return 0
        lax.fori_loop(0, NVEC // U, obody, 0)

    def in_copy(j, buf):
        return pltpu.make_async_copy(x_hbm.at[base + j], buf, isems[j % 3])

    def out_copy(j, buf):
        return pltpu.make_async_copy(buf, out_hbm.at[base + j], osems[j % 3])

    for m in range(3):
        in_copy(m, bufs[m]).start()

    for j in range(ROWS_PER_W):
        bj = bufs[j % 3]
        in_copy(j, bj).wait()
        T = compute_threshold(bj)
        if j == 1:
            out_copy(0, bufs[0]).wait()
            in_copy(3, bufs[0]).start()
        output_pass(bj, T)
        out_copy(j, bj).start()

    for j in (1, 2, 3):
        out_copy(j, bufs[j % 3]).wait()


@jax.jit
def _topk_mask(bits):
    mesh = plsc.VectorSubcoreMesh(core_axis_name="c", subcore_axis_name="s")
    f = functools.partial(
        pl.kernel,
        out_type=jax.ShapeDtypeStruct((B, N), jnp.int32),
        mesh=mesh,
        scratch_types=[
            pltpu.VMEM((N,), jnp.int32),
            pltpu.VMEM((N,), jnp.int32),
            pltpu.VMEM((N,), jnp.int32),
            pltpu.VMEM((NB,), jnp.int32),
            pltpu.VMEM((NB,), jnp.int32),
            pltpu.SemaphoreType.DMA,
            pltpu.SemaphoreType.DMA,
            pltpu.SemaphoreType.DMA,
            pltpu.SemaphoreType.DMA,
            pltpu.SemaphoreType.DMA,
            pltpu.SemaphoreType.DMA,
        ],
        compiler_params=pltpu.CompilerParams(needs_layout_passes=False),
    )(_topk_body)
    return f(bits)


def kernel(inputs):
    bits = lax.bitcast_convert_type(inputs, jnp.int32)
    out_bits = _topk_mask(bits)
    return lax.bitcast_convert_type(out_bits, jnp.float32)


# speedup vs baseline: 3.7932x; 1.0175x over previous
import functools

import jax
import jax.numpy as jnp
from jax import lax
from jax.experimental import pallas as pl
from jax.experimental.pallas import tpu as pltpu
from jax.experimental.pallas import tpu_sc as plsc

K = 256
B = 128
N = 32768
L = 16
NB = 2048
NCH = NB // L
NGRP = NCH // L
NVEC = N // L
NWORKERS = 32
ROWS_PER_W = B // NWORKERS
U = 16

NPASS = 3
SH_P = (31, 20, 9)
SH_D = (20, 9, 0)
DM = (2047, 2047, 511)
DBITS = (11, 11, 9)


def _sel(p, consts):
    v = jnp.int32(consts[-1])
    for q in range(len(consts) - 2, -1, -1):
        v = jnp.where(p == q, jnp.int32(consts[q]), v)
    return v


def _topk_body(x_hbm, out_hbm, b0, b1, b2, hist_v, scr_v,
               si0, si1, si2, so0, so1, so2):
    bufs = (b0, b1, b2)
    isems = (si0, si1, si2)
    osems = (so0, so1, so2)

    cid = lax.axis_index("c")
    sid = lax.axis_index("s")
    wid = sid * 2 + cid
    base = wid * ROWS_PER_W

    lane = lax.broadcasted_iota(jnp.int32, (L,), 0)
    lane16 = lane * L
    ones = jnp.ones((L,), jnp.int32)
    zeros = jnp.zeros((L,), jnp.int32)
    mask31 = jnp.int32(0x7FFFFFFF)

    def extract(vec, i):
        return jnp.sum(jnp.where(lane == i, vec, 0))

    def zero_hist():
        def zbody(i, _):
            for u in range(U):
                hist_v[pl.ds((i * U + u) * L, L)] = zeros
            return 0
        lax.fori_loop(0, NB // L // U, zbody, 0)

    def hist_pass1(row_v):
        def body(i, _):
            vs = [row_v[pl.ds((i * U + u) * L, L)] for u in range(U)]
            dd = [lax.bitwise_and(lax.shift_right_logical(v, 20), 2047)
                  for v in vs]
            for u in range(U):
                plsc.addupdate_scatter(hist_v, [dd[u]], ones)
            return 0
        lax.fori_loop(0, NVEC // U, body, 0)

    def hist_pass2(row_v, p1):
        def body(i, _):
            vs = [row_v[pl.ds((i * U + u) * L, L)] for u in range(U)]
            dd = [lax.bitwise_and(lax.shift_right_logical(v, 9), 2047)
                  for v in vs]
            mm = [lax.bitwise_and(lax.shift_right_logical(v, 20), 2047) == p1
                  for v in vs]
            for u in range(U):
                plsc.addupdate_scatter(hist_v, [dd[u]], ones, mask=mm[u])
            return 0
        lax.fori_loop(0, NVEC // U, body, 0)

    def hist_pass3(row_v, p12):
        def body(i, _):
            vs = [row_v[pl.ds((i * U + u) * L, L)] for u in range(U)]
            uu = [lax.shift_left(v, 1) for v in vs]
            dd = [lax.bitwise_and(v, 511) for v in vs]
            mm = [lax.shift_right_logical(u, 10) == p12 for u in uu]
            for u in range(U):
                plsc.addupdate_scatter(hist_v, [dd[u]], ones, mask=mm[u])
            return 0
        lax.fori_loop(0, NVEC // U, body, 0)

    def scan_hist(kin):
        def sbody(i, _):
            cs = [plsc.cumsum(lax.rev(hist_v[pl.ds((i * 8 + u) * L, L)], (0,)))
                  for u in range(8)]
            for u in range(8):
                scr_v[pl.ds((i * 8 + u) * L, L)] = cs[u]
            return 0
        lax.fori_loop(0, NCH // 8, sbody, 0)
        tots = [plsc.load_gather(scr_v, [lane16 + (g * 256 + (L - 1))])
                for g in range(NGRP)]
        csg = [plsc.cumsum(lax.rev(t, (0,))) for t in tots]
        gts = [jnp.max(c) for c in csg]
        cum = jnp.int32(0)
        found = jnp.int32(0)
        gstar = jnp.int32(0)
        need_g = jnp.int32(0)
        cs_g = zeros
        for g in range(NGRP - 1, -1, -1):
            hit = jnp.logical_and(found == 0, cum + gts[g] >= kin)
            hb = (zeros + hit.astype(jnp.int32)) == 1
            gstar = jnp.where(hit, g, gstar)
            need_g = jnp.where(hit, kin - cum, need_g)
            cs_g = jnp.where(hb, csg[g], cs_g)
            found = jnp.where(hit, 1, found)
            cum = cum + gts[g]
        q1 = jnp.max(plsc.all_reduce_ffs(cs_g >= need_g))
        tstar = gstar * L + (L - 1) - q1
        need_c = need_g - extract(cs_g, q1 - 1)
        cs_star = scr_v[pl.ds(tstar * L, L)]
        q2 = jnp.max(plsc.all_reduce_ffs(cs_star >= need_c))
        digit = tstar * L + (L - 1) - q2
        kin_next = need_c - extract(cs_star, q2 - 1)
        return digit, kin_next

    def compute_threshold(row_v):
        kin = jnp.int32(K)
        zero_hist()
        hist_pass1(row_v)
        p1, kin = scan_hist(kin)
        zero_hist()
        hist_pass2(row_v, p1)
        d2, kin = scan_hist(kin)
        p12 = lax.bitwise_or(lax.shift_left(p1, 11), d2)
        zero_hist()
        hist_pass3(row_v, p12)
        d3, _ = scan_hist(kin)
        return lax.bitwise_or(lax.shift_left(p12, 9), d3)

    def output_pass(row_v, T):
        def obody(i, _):
            vs = [row_v[pl.ds((i * U + u) * L, L)] for u in range(U)]
            aa = [lax.bitwise_and(v, mask31) for v in vs]
            oo = [jnp.where(a >= T, a, 0) for a in aa]
            for u in range(U):
                row_v[pl.ds((i * U + u) * L, L)] = oo[u]
            return 0
        lax.fori_loop(0, NVEC // U, obody, 0)

    def in_copy(j, buf):
        return pltpu.make_async_copy(x_hbm.at[base + j], buf, isems[j % 3])

    def out_copy(j, buf):
        return pltpu.make_async_copy(buf, out_hbm.at[base + j], osems[j % 3])

    for m in range(3):
        in_copy(m, bufs[m]).start()

    for j in range(ROWS_PER_W):
        bj = bufs[j % 3]
        in_copy(j, bj).wait()
        T = compute_threshold(bj)
        if j == 1:
            out_copy(0, bufs[0]).wait()
            in_copy(3, bufs[0]).start()
        output_pass(bj, T)
        out_copy(j, bj).start()

    for j in (1, 2, 3):
        out_copy(j, bufs[j % 3]).wait()


@jax.jit
def _topk_mask(bits):
    mesh = plsc.VectorSubcoreMesh(core_axis_name="c", subcore_axis_name="s")
    f = functools.partial(
        pl.kernel,
        out_type=jax.ShapeDtypeStruct((B, N), jnp.int32),
        mesh=mesh,
        scratch_types=[
            pltpu.VMEM((N,), jnp.int32),
            pltpu.VMEM((N,), jnp.int32),
            pltpu.VMEM((N,), jnp.int32),
            pltpu.VMEM((NB,), jnp.int32),
            pltpu.VMEM((NB,), jnp.int32),
            pltpu.SemaphoreType.DMA,
            pltpu.SemaphoreType.DMA,
            pltpu.SemaphoreType.DMA,
            pltpu.SemaphoreType.DMA,
            pltpu.SemaphoreType.DMA,
            pltpu.SemaphoreType.DMA,
        ],
        compiler_params=pltpu.CompilerParams(needs_layout_passes=False),
    )(_topk_body)
    return f(bits)


def kernel(inputs):
    bits = lax.bitcast_convert_type(inputs, jnp.int32)
    out_bits = _topk_mask(bits)
    return lax.bitcast_convert_type(out_bits, jnp.float32)
